# Initial kernel scaffold; baseline (speedup 1.0000x reference)
import jax
import jax.numpy as jnp
from jax.experimental import pallas as pl


def kernel(features, adj_indices, adj_values, weight):
    raise NotImplementedError("write your pallas kernel here")


# speedup vs baseline: 5.5375x; 5.5375x over previous
import functools

import jax
import jax.numpy as jnp
from jax import lax
from jax.experimental import pallas as pl
from jax.experimental.pallas import tpu as pltpu
from jax.experimental.pallas import tpu_sc as plsc

N_NODES = 10000
FDIM = 128
CHUNK = 128
NC = 2
NS = 16
NW = NC * NS
ROWS_MAIN = (N_NODES // NS) // 8 * 8
ROWS_TAIL = N_NODES - NS * ROWS_MAIN


def _sc_aggregate(row, col, vals, features):
    n_edges = row.shape[0]
    n_chunks = n_edges // CHUNK
    full_rounds = n_chunks // NW
    rem = n_chunks - full_rounds * NW

    mesh = plsc.VectorSubcoreMesh(core_axis_name="c", subcore_axis_name="s")

    @functools.partial(
        pl.kernel,
        mesh=mesh,
        out_type=jax.ShapeDtypeStruct((NC, N_NODES, FDIM), jnp.float32),
        scratch_types=[
            pltpu.VMEM_SHARED((N_NODES, FDIM), jnp.float32),
            pltpu.VMEM((1, CHUNK), jnp.int32),
            pltpu.VMEM((1, CHUNK), jnp.int32),
            pltpu.VMEM((CHUNK,), jnp.float32),
            pltpu.VMEM((CHUNK, FDIM), jnp.float32),
            pltpu.SemaphoreType.DMA,
        ],
    )
    def agg(row_hbm, col_hbm, val_hbm, feat_hbm, out_hbm, acc, colv, rowv,
            valv, grows, sem):
        cc = lax.axis_index("c")
        sid = lax.axis_index("s")
        wid = sid * NC + cc

        def zrow(r, _):
            def zcol(i, _):
                grows[r, pl.ds(i * 16, 16)] = jnp.zeros((16,), jnp.float32)
                return 0
            return lax.fori_loop(0, FDIM // 16, zcol, 0)
        lax.fori_loop(0, CHUNK, zrow, 0)

        base_row = sid * ROWS_MAIN
        for j in range(ROWS_MAIN // CHUNK):
            pltpu.sync_copy(grows, acc.at[pl.ds(base_row + j * CHUNK, CHUNK)])
        tail = ROWS_MAIN - (ROWS_MAIN // CHUNK) * CHUNK
        pltpu.sync_copy(
            grows.at[pl.ds(0, tail)],
            acc.at[pl.ds(base_row + (ROWS_MAIN // CHUNK) * CHUNK, tail)])

        @pl.when(sid == 0)
        def _():
            pltpu.sync_copy(grows.at[pl.ds(0, ROWS_TAIL)],
                            acc.at[pl.ds(NS * ROWS_MAIN, ROWS_TAIL)])
        plsc.subcore_barrier()

        my_chunks = full_rounds + jnp.where(wid < rem, 1, 0)

        def chunk_body(g, _):
            cid = g * NW + wid
            base = cid * CHUNK
            pltpu.sync_copy(col_hbm.at[pl.ds(base, CHUNK)], colv.at[0])
            pltpu.sync_copy(row_hbm.at[pl.ds(base, CHUNK)], rowv.at[0])
            pltpu.sync_copy(val_hbm.at[pl.ds(base, CHUNK)], valv)
            pltpu.async_copy(feat_hbm.at[colv.at[0]], grows, sem).wait()

            def grp_body(grp, _):
                vv = valv[pl.ds(grp * 16, 16)]
                for lane in range(16):
                    v = vv[lane]
                    e = grp * 16 + lane
                    for kk in range(FDIM // 16):
                        grows[e, pl.ds(kk * 16, 16)] = (
                            grows[e, pl.ds(kk * 16, 16)] * v)
                return 0
            lax.fori_loop(0, CHUNK // 16, grp_body, 0)

            pltpu.sync_copy(grows, acc.at[rowv.at[0]], add=True)
            return 0

        lax.fori_loop(0, my_chunks, chunk_body, 0)
        plsc.subcore_barrier()

        pltpu.sync_copy(acc.at[pl.ds(base_row, ROWS_MAIN)],
                        out_hbm.at[cc, pl.ds(base_row, ROWS_MAIN)])

        @pl.when(sid == 0)
        def _():
            pltpu.sync_copy(acc.at[pl.ds(NS * ROWS_MAIN, ROWS_TAIL)],
                            out_hbm.at[cc, pl.ds(NS * ROWS_MAIN, ROWS_TAIL)])

    return agg(row, col, vals, features)


def _tc_combine_matmul(partials, weight):
    bn = 1000

    def body(p_ref, w_ref, o_ref):
        s = p_ref[0] + p_ref[1]
        o_ref[...] = jnp.maximum(
            jnp.dot(s, w_ref[...], preferred_element_type=jnp.float32), 0.0)

    return pl.pallas_call(
        body,
        grid=(N_NODES // bn,),
        in_specs=[
            pl.BlockSpec((NC, bn, FDIM), lambda i: (0, i, 0)),
            pl.BlockSpec((FDIM, FDIM), lambda i: (0, 0)),
        ],
        out_specs=pl.BlockSpec((bn, FDIM), lambda i: (i, 0)),
        out_shape=jax.ShapeDtypeStruct((N_NODES, FDIM), jnp.float32),
    )(partials, weight)


def kernel(features, adj_indices, adj_values, weight):
    idx = adj_indices.astype(jnp.int32)
    partials = _sc_aggregate(idx[0], idx[1], adj_values, features)
    return _tc_combine_matmul(partials, weight)
